# baseline (speedup 1.0000x reference)
import functools

import jax
import jax.numpy as jnp
from jax import lax
from jax.experimental import pallas as pl
from jax.experimental.pallas import tpu as pltpu
from jax.experimental.pallas import tpu_sc as plsc

_N_USERS = 25000
_N = 50000
_D = 32
_E = 1600000
_NPAD = 51200
_EPAD = 1605632
_GPW = 392
_K = 8
_CHUNKS = _GPW // _K
_C = _K * 128
_RPT = _NPAD // 16

_MESH = plsc.VectorSubcoreMesh(core_axis_name="c", subcore_axis_name="s")

_GDN = lax.GatherDimensionNumbers(
    offset_dims=(), collapsed_slice_dims=(0,), start_index_map=(0,)
)


def _dg(x, idx16):
    return lax.gather(
        x,
        idx16.reshape(16, 1),
        _GDN,
        (1,),
        mode=lax.GatherScatterMode.PROMISE_IN_BOUNDS,
    )


@functools.partial(
    pl.kernel,
    out_type=[
        jax.ShapeDtypeStruct((_NPAD, _D), jnp.float32),
        jax.ShapeDtypeStruct((_NPAD, _D), jnp.float32),
    ],
    mesh=_MESH,
    compiler_params=pltpu.CompilerParams(use_tc_tiling_on_sc=False),
    scratch_types=[
        pltpu.VMEM((2, _K, 128), jnp.int32),
        pltpu.VMEM((2, _K, 128), jnp.int32),
        pltpu.VMEM((2, _C), jnp.float32),
        pltpu.VMEM((4, 128, _D), jnp.bfloat16),
        pltpu.VMEM((2, 128, _D), jnp.float32),
        pltpu.VMEM_SHARED((_NPAD, _D), jnp.float32),
    ]
    + [pltpu.SemaphoreType.DMA] * 8,
)
def _prop(
    table, srcg, dstg, valf, p0, p1, src_v, dst_v, val_v, rows_bf, srows, acc,
    g0, g1, g2, g3, s0, s1, stsem, zsem,
):
    cid = lax.axis_index("c")
    sid = lax.axis_index("s")
    wid = sid * 2 + cid
    gsem = [g0, g1, g2, g3]
    ssem = [s0, s1]

    def _scat_wait(bs):
        pltpu.make_async_copy(
            p0.at[pl.ds(0, 128)], srows.at[bs], ssem[bs]
        ).wait()

    def _stage(c):
        pc = lax.rem(c, 2)
        rown = wid * _GPW + c * _K
        pltpu.async_copy(srcg.at[pl.ds(rown, _K)], src_v.at[pc], stsem)
        pltpu.async_copy(dstg.at[pl.ds(rown, _K)], dst_v.at[pc], stsem)
        pltpu.async_copy(valf.at[pl.ds(rown * 128, _C)], val_v.at[pc], stsem)

    def _stage_wait(pc):
        pltpu.make_async_copy(srcg.at[pl.ds(0, _K)], src_v.at[pc], stsem).wait()
        pltpu.make_async_copy(dstg.at[pl.ds(0, _K)], dst_v.at[pc], stsem).wait()
        pltpu.make_async_copy(valf.at[pl.ds(0, _C)], val_v.at[pc], stsem).wait()

    def _zrow(i, carry):
        z = jnp.zeros((16,), jnp.float32)
        srows[0, i, pl.ds(0, 16)] = z
        srows[0, i, pl.ds(16, 16)] = z
        return carry

    lax.fori_loop(0, 128, _zrow, 0)
    zdescs = [
        pltpu.async_copy(
            srows.at[0], acc.at[pl.ds(sid * _RPT + h * 128, 128)], zsem
        )
        for h in range(_RPT // 128)
    ]
    for dsc in zdescs:
        dsc.wait()
    plsc.subcore_barrier()

    _stage(0)
    _stage_wait(0)
    for j in range(4):
        pltpu.async_copy(table.at[src_v.at[0, j]], rows_bf.at[j], gsem[j])

    def _scale(b, bs, voff, val_p):
        @plsc.parallel_loop(0, 8, unroll=2)
        def _sg(g):
            v16 = val_p[pl.ds(voff + g * 16, 16)]
            for i in range(16):
                e = g * 16 + i
                sp = _dg(v16, jnp.full((16,), i, jnp.int32))
                rb = rows_bf[b, e, pl.ds(0, _D)].astype(jnp.float32)
                srows[bs, e, pl.ds(0, 16)] = lax.slice(rb, (0,), (16,)) * sp
                srows[bs, e, pl.ds(16, 16)] = lax.slice(rb, (16,), (32,)) * sp

    def _chunk(c, carry):
        p = lax.rem(c, 2)
        pn = lax.rem(c + 1, 2)
        last = c == _CHUNKS - 1

        @pl.when(jnp.logical_not(last))
        def _():
            _stage(c + 1)

        sdescs = [None] * _K
        for j in range(_K):
            b = j % 4
            bs = j % 2
            pltpu.make_async_copy(
                table.at[pl.ds(0, 128)], rows_bf.at[b], gsem[b]
            ).wait()
            if j < 2:

                @pl.when(c > 0)
                def _():
                    _scat_wait(bs)

            else:
                sdescs[j - 2].wait()
            _scale(b, bs, j * 128, val_v.at[p])
            sdescs[j] = pltpu.async_copy(
                srows.at[bs], acc.at[dst_v.at[p, j]], ssem[bs], add=True
            )
            if j < 4:
                pltpu.async_copy(
                    table.at[src_v.at[p, j + 4]], rows_bf.at[b], gsem[b]
                )
            else:
                if j == 4:

                    @pl.when(jnp.logical_not(last))
                    def _():
                        _stage_wait(pn)

                @pl.when(jnp.logical_not(last))
                def _():
                    pltpu.async_copy(
                        table.at[src_v.at[pn, j - 4]], rows_bf.at[b], gsem[b]
                    )

        return carry

    lax.fori_loop(0, _CHUNKS, _chunk, 0)
    for bs in range(2):
        _scat_wait(bs)

    plsc.subcore_barrier()
    r0 = sid * _RPT

    @pl.when(cid == 0)
    def _():
        pltpu.sync_copy(acc.at[pl.ds(r0, _RPT)], p0.at[pl.ds(r0, _RPT)])

    @pl.when(cid == 1)
    def _():
        pltpu.sync_copy(acc.at[pl.ds(r0, _RPT)], p1.at[pl.ds(r0, _RPT)])


def _merge_body(p0_ref, p1_ref, s_ref, t_out, s_out):
    t = p0_ref[...] + p1_ref[...]
    t_out[...] = t.astype(jnp.bfloat16)
    s_out[...] = s_ref[...] + t


def _merge(p0, p1, s):
    rows = _NPAD * _D // 128
    blk = rows // 8
    f = pl.pallas_call(
        _merge_body,
        out_shape=[
            jax.ShapeDtypeStruct((rows, 128), jnp.bfloat16),
            jax.ShapeDtypeStruct((rows, 128), jnp.float32),
        ],
        grid=(8,),
        in_specs=[pl.BlockSpec((blk, 128), lambda i: (i, 0))] * 3,
        out_specs=[pl.BlockSpec((blk, 128), lambda i: (i, 0))] * 2,
    )
    tbf, s2 = f(
        p0.reshape(rows, 128), p1.reshape(rows, 128), s.reshape(rows, 128)
    )
    return tbf.reshape(_NPAD, _D), s2.reshape(_NPAD, _D)


@functools.partial(
    pl.kernel,
    out_type=[
        jax.ShapeDtypeStruct((4096, _D), jnp.float32),
        jax.ShapeDtypeStruct((4096, _D), jnp.float32),
    ],
    mesh=_MESH,
    compiler_params=pltpu.CompilerParams(use_tc_tiling_on_sc=False),
    scratch_types=[
        pltpu.VMEM((128,), jnp.int32),
        pltpu.VMEM((128,), jnp.int32),
        pltpu.VMEM((128, _D), jnp.float32),
        pltpu.VMEM((128, _D), jnp.float32),
        pltpu.SemaphoreType.DMA,
    ],
)
def _gather2(sum_t, uid, gid, ur_o, ir_o, uid_v, gid_v, ur_v, ir_v, sem):
    cid = lax.axis_index("c")
    sid = lax.axis_index("s")
    base = (sid * 2 + cid) * 128
    pltpu.sync_copy(uid.at[pl.ds(base, 128)], uid_v)
    pltpu.sync_copy(gid.at[pl.ds(base, 128)], gid_v)
    a = pltpu.async_copy(sum_t.at[uid_v], ur_v, sem)
    b = pltpu.async_copy(sum_t.at[gid_v], ir_v, sem)
    a.wait()
    b.wait()
    pltpu.sync_copy(ur_v, ur_o.at[pl.ds(base, 128)])
    pltpu.sync_copy(ir_v, ir_o.at[pl.ds(base, 128)])


def _dot_body(u_ref, i_ref, o_ref):
    o_ref[...] = jnp.sum(u_ref[...] * i_ref[...], axis=1) * jnp.float32(1.0 / 16.0)


def kernel(users, items, user_emb, item_emb, edge_src, edge_dst, edge_val):
    table0 = (
        jnp.zeros((_NPAD, _D), jnp.float32)
        .at[:_N]
        .set(jnp.concatenate([user_emb, item_emb], axis=0))
    )
    pad = _EPAD - _E
    srcg = jnp.concatenate([edge_src, jnp.zeros((pad,), jnp.int32)]).reshape(
        _EPAD // 128, 128
    )
    dstg = jnp.concatenate([edge_dst, jnp.zeros((pad,), jnp.int32)]).reshape(
        _EPAD // 128, 128
    )
    valf = jnp.concatenate([edge_val, jnp.zeros((pad,), jnp.float32)])
    tb = table0.astype(jnp.bfloat16)
    s = table0
    for _ in range(3):
        p0, p1 = _prop(tb, srcg, dstg, valf)
        tb, s = _merge(p0, p1, s)
    ur, ir = _gather2(s, users, items + jnp.int32(_N_USERS))
    return pl.pallas_call(
        _dot_body, out_shape=jax.ShapeDtypeStruct((4096,), jnp.float32)
    )(ur, ir)


# speedup vs baseline: 1.0303x; 1.0303x over previous
import functools

import jax
import jax.numpy as jnp
from jax import lax
from jax.experimental import pallas as pl
from jax.experimental.pallas import tpu as pltpu
from jax.experimental.pallas import tpu_sc as plsc

_N_USERS = 25000
_N = 50000
_D = 32
_E = 1600000
_NPAD = 51200
_EPAD = 1605632
_GPW = 392
_K = 8
_CHUNKS = _GPW // _K
_C = _K * 128
_RPT = _NPAD // 16

_MESH = plsc.VectorSubcoreMesh(core_axis_name="c", subcore_axis_name="s")

_GDN = lax.GatherDimensionNumbers(
    offset_dims=(), collapsed_slice_dims=(0,), start_index_map=(0,)
)


def _dg(x, idx16):
    return lax.gather(
        x,
        idx16.reshape(16, 1),
        _GDN,
        (1,),
        mode=lax.GatherScatterMode.PROMISE_IN_BOUNDS,
    )


@functools.partial(
    pl.kernel,
    out_type=[
        jax.ShapeDtypeStruct((_NPAD, _D), jnp.float32),
        jax.ShapeDtypeStruct((_NPAD, _D), jnp.float32),
    ],
    mesh=_MESH,
    compiler_params=pltpu.CompilerParams(use_tc_tiling_on_sc=False),
    scratch_types=[
        pltpu.VMEM((2, _K, 128), jnp.int32),
        pltpu.VMEM((2, _K, 128), jnp.int32),
        pltpu.VMEM((2, _C), jnp.float32),
        pltpu.VMEM((4, 128, _D), jnp.float32),
        pltpu.VMEM_SHARED((_NPAD, _D), jnp.float32),
    ]
    + [pltpu.SemaphoreType.DMA] * 10,
)
def _prop(
    table, srcg, dstg, valf, p0, p1, src_v, dst_v, val_v, rows_v, acc,
    g0, g1, g2, g3, s0, s1, s2, s3, stsem, zsem,
):
    cid = lax.axis_index("c")
    sid = lax.axis_index("s")
    wid = sid * 2 + cid
    gsem = [g0, g1, g2, g3]
    ssem = [s0, s1, s2, s3]

    def _scat_wait(b):
        pltpu.make_async_copy(
            table.at[pl.ds(0, 128)], rows_v.at[b], ssem[b]
        ).wait()

    def _zrow(i, carry):
        z = jnp.zeros((16,), jnp.float32)
        rows_v[0, i, pl.ds(0, 16)] = z
        rows_v[0, i, pl.ds(16, 16)] = z
        return carry

    lax.fori_loop(0, 128, _zrow, 0)
    zdescs = [
        pltpu.async_copy(
            rows_v.at[0], acc.at[pl.ds(sid * _RPT + h * 128, 128)], zsem
        )
        for h in range(_RPT // 128)
    ]
    for dsc in zdescs:
        dsc.wait()
    plsc.subcore_barrier()

    row00 = wid * _GPW
    pltpu.async_copy(srcg.at[pl.ds(row00, _K)], src_v.at[0], stsem)
    pltpu.async_copy(dstg.at[pl.ds(row00, _K)], dst_v.at[0], stsem)
    pltpu.async_copy(valf.at[pl.ds(row00 * 128, _C)], val_v.at[0], stsem)

    def _scale(b, voff, val_p):
        @plsc.parallel_loop(0, 8, unroll=2)
        def _sg(g):
            v16 = val_p[pl.ds(voff + g * 16, 16)]
            for i in range(16):
                e = g * 16 + i
                sp = _dg(v16, jnp.full((16,), i, jnp.int32))
                rows_v[b, e, pl.ds(0, 16)] = rows_v[b, e, pl.ds(0, 16)] * sp
                rows_v[b, e, pl.ds(16, 16)] = rows_v[b, e, pl.ds(16, 16)] * sp

    def _chunk(c, carry):
        p = lax.rem(c, 2)
        pltpu.make_async_copy(srcg.at[pl.ds(0, _K)], src_v.at[p], stsem).wait()
        pltpu.make_async_copy(dstg.at[pl.ds(0, _K)], dst_v.at[p], stsem).wait()
        pltpu.make_async_copy(valf.at[pl.ds(0, _C)], val_v.at[p], stsem).wait()

        @pl.when(c < _CHUNKS - 1)
        def _():
            cn = c + 1
            pn = lax.rem(cn, 2)
            rown = wid * _GPW + cn * _K
            pltpu.async_copy(srcg.at[pl.ds(rown, _K)], src_v.at[pn], stsem)
            pltpu.async_copy(dstg.at[pl.ds(rown, _K)], dst_v.at[pn], stsem)
            pltpu.async_copy(valf.at[pl.ds(rown * 128, _C)], val_v.at[pn], stsem)

        @pl.when(c > 0)
        def _():
            _scat_wait(0)
            _scat_wait(1)

        gdescs = [None] * _K
        sdescs = [None] * _K
        for j in range(2):
            gdescs[j] = pltpu.async_copy(
                table.at[src_v.at[p, j]], rows_v.at[j], gsem[j]
            )
        for j in range(_K):
            b = j % 4
            if j < _K - 2:
                b2 = (j + 2) % 4
                if j + 2 < 4:

                    @pl.when(c > 0)
                    def _():
                        _scat_wait(b2)

                else:
                    sdescs[j - 2].wait()
                gdescs[j + 2] = pltpu.async_copy(
                    table.at[src_v.at[p, j + 2]], rows_v.at[b2], gsem[b2]
                )
            gdescs[j].wait()
            _scale(b, j * 128, val_v.at[p])
            sdescs[j] = pltpu.async_copy(
                rows_v.at[b], acc.at[dst_v.at[p, j]], ssem[b], add=True
            )
        return carry

    lax.fori_loop(0, _CHUNKS, _chunk, 0)
    for b in range(4):
        _scat_wait(b)

    plsc.subcore_barrier()
    r0 = sid * _RPT

    @pl.when(cid == 0)
    def _():
        pltpu.sync_copy(acc.at[pl.ds(r0, _RPT)], p0.at[pl.ds(r0, _RPT)])

    @pl.when(cid == 1)
    def _():
        pltpu.sync_copy(acc.at[pl.ds(r0, _RPT)], p1.at[pl.ds(r0, _RPT)])


def _merge_body(p0_ref, p1_ref, s_ref, t_out, s_out):
    t = p0_ref[...] + p1_ref[...]
    t_out[...] = t
    s_out[...] = s_ref[...] + t


def _merge(p0, p1, s):
    rows = _NPAD * _D // 128
    blk = rows // 8
    f = pl.pallas_call(
        _merge_body,
        out_shape=[jax.ShapeDtypeStruct((rows, 128), jnp.float32)] * 2,
        grid=(8,),
        in_specs=[pl.BlockSpec((blk, 128), lambda i: (i, 0))] * 3,
        out_specs=[pl.BlockSpec((blk, 128), lambda i: (i, 0))] * 2,
    )
    t, s2 = f(
        p0.reshape(rows, 128), p1.reshape(rows, 128), s.reshape(rows, 128)
    )
    return t.reshape(_NPAD, _D), s2.reshape(_NPAD, _D)


@functools.partial(
    pl.kernel,
    out_type=[
        jax.ShapeDtypeStruct((4096, _D), jnp.float32),
        jax.ShapeDtypeStruct((4096, _D), jnp.float32),
    ],
    mesh=_MESH,
    compiler_params=pltpu.CompilerParams(use_tc_tiling_on_sc=False),
    scratch_types=[
        pltpu.VMEM((128,), jnp.int32),
        pltpu.VMEM((128,), jnp.int32),
        pltpu.VMEM((128, _D), jnp.float32),
        pltpu.VMEM((128, _D), jnp.float32),
        pltpu.SemaphoreType.DMA,
    ],
)
def _gather2(sum_t, uid, gid, ur_o, ir_o, uid_v, gid_v, ur_v, ir_v, sem):
    cid = lax.axis_index("c")
    sid = lax.axis_index("s")
    base = (sid * 2 + cid) * 128
    pltpu.sync_copy(uid.at[pl.ds(base, 128)], uid_v)
    pltpu.sync_copy(gid.at[pl.ds(base, 128)], gid_v)
    a = pltpu.async_copy(sum_t.at[uid_v], ur_v, sem)
    b = pltpu.async_copy(sum_t.at[gid_v], ir_v, sem)
    a.wait()
    b.wait()
    pltpu.sync_copy(ur_v, ur_o.at[pl.ds(base, 128)])
    pltpu.sync_copy(ir_v, ir_o.at[pl.ds(base, 128)])


def _dot_body(u_ref, i_ref, o_ref):
    o_ref[...] = jnp.sum(u_ref[...] * i_ref[...], axis=1) * jnp.float32(1.0 / 16.0)


def kernel(users, items, user_emb, item_emb, edge_src, edge_dst, edge_val):
    table0 = (
        jnp.zeros((_NPAD, _D), jnp.float32)
        .at[:_N]
        .set(jnp.concatenate([user_emb, item_emb], axis=0))
    )
    pad = _EPAD - _E
    srcg = jnp.concatenate([edge_src, jnp.zeros((pad,), jnp.int32)]).reshape(
        _EPAD // 128, 128
    )
    dstg = jnp.concatenate([edge_dst, jnp.zeros((pad,), jnp.int32)]).reshape(
        _EPAD // 128, 128
    )
    valf = jnp.concatenate([edge_val, jnp.zeros((pad,), jnp.float32)])
    table = table0
    s = table0
    for _ in range(3):
        p0, p1 = _prop(table, srcg, dstg, valf)
        table, s = _merge(p0, p1, s)
    ur, ir = _gather2(s, users, items + jnp.int32(_N_USERS))
    return pl.pallas_call(
        _dot_body, out_shape=jax.ShapeDtypeStruct((4096,), jnp.float32)
    )(ur, ir)
